# baseline (speedup 1.0000x reference)
import functools

import jax
import jax.numpy as jnp
from jax import lax
from jax.experimental import pallas as pl
from jax.experimental.pallas import tpu as pltpu
from jax.experimental.pallas import tpu_sc as plsc

_N = 50000
_E = 800000
_H = 64
_RBF = 50
_G = 64
_OUT = 64
_NI = 3
_GAMMA = 10.0

_CH = 88
_NCH = 9216
_EP = _NCH * _CH
_TE = 2048
_TN = 2000
_NPAD = 50016
_NCH0 = 5760
_NT0 = _NCH0 // 16
_NT1 = (_NCH - _NCH0) // 16
_ZROWS = _NPAD // 16
_OCH = 400
_NOCH = _N // _OCH

_PACK_PERM = [0] * _H
for _t in range(16):
    for _J in range(2):
        _PACK_PERM[32 * _J + 2 * _t] = 32 * _J + _t
        _PACK_PERM[32 * _J + 2 * _t + 1] = 32 * _J + 16 + _t
del _t, _J


def _silu(x):
    return x * jax.nn.sigmoid(x)


def _embed_body(x_ref, w_ref, b_ref, o_ref):
    t = jnp.dot(x_ref[...], w_ref[...], preferred_element_type=jnp.float32)
    o_ref[...] = _silu(t + b_ref[...])


def _embed(x, W0, b0):
    return pl.pallas_call(
        _embed_body,
        grid=(_N // _TN,),
        in_specs=[
            pl.BlockSpec((_TN, 3), lambda b: (b, 0)),
            pl.BlockSpec((3, _H), lambda b: (0, 0)),
            pl.BlockSpec((1, _H), lambda b: (0, 0)),
        ],
        out_specs=pl.BlockSpec((_TN, _H), lambda b: (b, 0)),
        out_shape=jax.ShapeDtypeStruct((_N, _H), jnp.float32),
    )(x, W0, b0)


def _filter_body(d_ref, wf1_ref, bf1_ref, wf2_ref, bf2_ref, o_ref):
    d2 = d_ref[...]
    col = lax.broadcasted_iota(jnp.int32, (1, 2 * _RBF), 1)
    centers = (col % _RBF).astype(jnp.float32) * (6.0 / (_RBF - 1))
    dsel = jnp.where(col >= _RBF, d2[:, 1:2], d2[:, 0:1])
    diff = dsel - centers
    rbf = jnp.exp(-_GAMMA * diff * diff)
    t = jnp.dot(rbf, wf1_ref[...], preferred_element_type=jnp.float32) + bf1_ref[...]
    t = _silu(t)
    o_ref[...] = (
        jnp.dot(t, wf2_ref[...], preferred_element_type=jnp.float32) + bf2_ref[...]
    )


def _filters_i(edge_dist, Wf1d, bf1d, Wf2d, bf2d):
    te2 = _TE // 2
    grid = (_E + _TE - 1) // _TE
    return pl.pallas_call(
        _filter_body,
        grid=(grid,),
        in_specs=[
            pl.BlockSpec((te2, 2), lambda b: (b, 0)),
            pl.BlockSpec((2 * _RBF, 2 * _H), lambda b: (0, 0)),
            pl.BlockSpec((1, 2 * _H), lambda b: (0, 0)),
            pl.BlockSpec((2 * _H, 2 * _H), lambda b: (0, 0)),
            pl.BlockSpec((1, 2 * _H), lambda b: (0, 0)),
        ],
        out_specs=pl.BlockSpec((te2, 2 * _H), lambda b: (b, 0)),
        out_shape=jax.ShapeDtypeStruct((_EP // 2, 2 * _H), jnp.float32),
    )(edge_dist.reshape(_E // 2, 2), Wf1d, bf1d, Wf2d, bf2d)


def _filter_weights(Wf1, bf1, Wf2, bf2):
    Wf1d = jnp.zeros((_NI, 2 * _RBF, 2 * _H), jnp.float32)
    Wf1d = Wf1d.at[:, :_RBF, :_H].set(Wf1).at[:, _RBF:, _H:].set(Wf1)
    Wf2d = jnp.zeros((_NI, 2 * _H, 2 * _H), jnp.float32)
    Wf2d = Wf2d.at[:, :_H, :_H].set(Wf2).at[:, _H:, _H:].set(Wf2)
    bf1d = jnp.tile(bf1, (1, 2)).reshape(_NI, 1, 2 * _H)
    bf2d = jnp.tile(bf2, (1, 2)).reshape(_NI, 1, 2 * _H)
    return Wf1d, bf1d, Wf2d, bf2d


@functools.lru_cache(maxsize=1)
def _make_sc_msg():
    mesh = plsc.VectorSubcoreMesh(core_axis_name="c", subcore_axis_name="s")

    @functools.partial(
        pl.kernel,
        out_type=[jax.ShapeDtypeStruct((_N, _H), jnp.bfloat16),
                  jax.ShapeDtypeStruct((_N, _H), jnp.bfloat16)],
        mesh=mesh,
        scratch_types=[
            pltpu.VMEM((4, _CH), jnp.int32),
            pltpu.VMEM((4, _CH), jnp.int32),
            pltpu.VMEM((2, _CH, _H), jnp.float32),
            pltpu.VMEM((2, _CH // 2, 2 * _H), jnp.float32),
            pltpu.VMEM((2, _CH, _H), jnp.bfloat16),
            pltpu.VMEM_SHARED((_NPAD, _H), jnp.bfloat16),
            pltpu.SemaphoreType.DMA, pltpu.SemaphoreType.DMA,
            pltpu.SemaphoreType.DMA, pltpu.SemaphoreType.DMA,
            pltpu.SemaphoreType.DMA, pltpu.SemaphoreType.DMA,
            pltpu.SemaphoreType.DMA, pltpu.SemaphoreType.DMA,
            pltpu.SemaphoreType.DMA, pltpu.SemaphoreType.DMA,
        ],
        compiler_params=pltpu.CompilerParams(use_tc_tiling_on_sc=False,
                                             needs_layout_passes=False),
    )
    def _sc_msg(h_hbm, fil_hbm, src_hbm, dst_hbm, agg0_hbm, agg1_hbm,
                src_v, dst_v, rows_v, fil_v, msg_v, acc_sh,
                is0, is1, is2, is3, fs0, fs1, gs0, gs1, ss0, ss1):
        c = lax.axis_index("c")
        s = lax.axis_index("s")
        isem = (is0, is1, is2, is3)
        fsem = (fs0, fs1)
        gsem = (gs0, gs1)
        ssem = (ss0, ss1)

        nt = jnp.where(c == 0, _NT0, _NT1)

        def chunk_of(i):
            return c * _NCH0 + s + 16 * i

        def idx_load(b4, i):
            base = chunk_of(i) * _CH
            pltpu.async_copy(src_hbm.at[pl.ds(base, _CH)], src_v.at[b4], isem[b4])
            pltpu.async_copy(dst_hbm.at[pl.ds(base, _CH)], dst_v.at[b4], isem[b4])

        def idx_wait(b4):
            pltpu.make_async_copy(src_hbm.at[pl.ds(0, _CH)], src_v.at[b4], isem[b4]).wait()
            pltpu.make_async_copy(dst_hbm.at[pl.ds(0, _CH)], dst_v.at[b4], isem[b4]).wait()

        def fil_load(b2, i):
            fb = chunk_of(i) * (_CH // 2)
            pltpu.async_copy(fil_hbm.at[pl.ds(fb, _CH // 2)], fil_v.at[b2], fsem[b2])

        def fil_wait(b2):
            pltpu.make_async_copy(fil_hbm.at[pl.ds(0, _CH // 2)], fil_v.at[b2],
                                  fsem[b2]).wait()

        def scat_wait(b2, b4):
            pltpu.make_async_copy(msg_v.at[b2], acc_sh.at[dst_v.at[b4]],
                                  ssem[b2]).wait()

        def fire_gather(b2, b4):
            pltpu.async_copy(h_hbm.at[src_v.at[b4]], rows_v.at[b2], gsem[b2])

        def compute(b2, b4):
            fil_wait(b2)
            pltpu.make_async_copy(h_hbm.at[src_v.at[b4]], rows_v.at[b2],
                                  gsem[b2]).wait()

            @pl.loop(0, _CH // 2, unroll=2)
            def _mul(rr):
                for half in range(2):
                    r = 2 * rr + half
                    p = [rows_v[b2, r, pl.ds(j * 16, 16)]
                         * fil_v[b2, rr, pl.ds(half * _H + j * 16, 16)]
                         for j in range(4)]
                    msg_v[b2, r, pl.ds(0, 32)] = plsc.pack(
                        p[0], p[1], format=plsc.PackFormat.INTERLEAVED)
                    msg_v[b2, r, pl.ds(32, 32)] = plsc.pack(
                        p[2], p[3], format=plsc.PackFormat.INTERLEAVED)

            pltpu.async_copy(msg_v.at[b2], acc_sh.at[dst_v.at[b4]], ssem[b2],
                             add=True)

        idx_load(0, 0)
        idx_load(1, 1)
        fil_load(0, 0)

        @pl.loop(0, _CH)
        def _zero_rows(r):
            msg_v[1, r, pl.ds(0, 32)] = jnp.zeros((32,), jnp.bfloat16)
            msg_v[1, r, pl.ds(32, 32)] = jnp.zeros((32,), jnp.bfloat16)

        @pl.loop(0, _ZROWS // _CH)
        def _zero_acc(k):
            pltpu.sync_copy(msg_v.at[1],
                            acc_sh.at[pl.ds(s * _ZROWS + k * _CH, _CH)])
        pltpu.sync_copy(
            msg_v.at[1, pl.ds(0, _ZROWS % _CH)],
            acc_sh.at[pl.ds(s * _ZROWS + (_ZROWS // _CH) * _CH, _ZROWS % _CH)])

        plsc.subcore_barrier()

        idx_wait(0)
        fire_gather(0, 0)

        @pl.loop(0, nt // 4)
        def _main(k):
            i0 = 4 * k
            for u in range(4):
                i = i0 + u
                b2, b4 = u % 2, u % 4
                nb2, nb4 = (u + 1) % 2, (u + 1) % 4

                @pl.when(i + 2 < nt)
                def _():
                    idx_load((u + 2) % 4, i + 2)

                @pl.when(i + 1 < nt)
                def _():
                    fil_load(nb2, i + 1)
                    idx_wait(nb4)

                    @pl.when(i >= 1)
                    def _():
                        scat_wait(nb2, nb4)

                    fire_gather(nb2, nb4)

                compute(b2, b4)

        scat_wait(0, 0)
        scat_wait(1, 1)
        plsc.subcore_barrier()

        @pl.when(c == 0)
        def _():
            @pl.loop(s, _NOCH, step=16)
            def _out(k):
                pltpu.sync_copy(acc_sh.at[pl.ds(k * _OCH, _OCH)],
                                agg0_hbm.at[pl.ds(k * _OCH, _OCH)])

        @pl.when(c == 1)
        def _():
            @pl.loop(s, _NOCH, step=16)
            def _out(k):
                pltpu.sync_copy(acc_sh.at[pl.ds(k * _OCH, _OCH)],
                                agg1_hbm.at[pl.ds(k * _OCH, _OCH)])

    return _sc_msg


def _update_body(h_ref, a0_ref, a1_ref, w_ref, b_ref, o_ref):
    a = a0_ref[...].astype(jnp.float32) + a1_ref[...].astype(jnp.float32)
    t = jnp.dot(a, w_ref[...], preferred_element_type=jnp.float32)
    o_ref[...] = h_ref[...] + _silu(t + b_ref[...])


def _update(h, a0, a1, Wl_eff, bl):
    return pl.pallas_call(
        _update_body,
        grid=(_N // _TN,),
        in_specs=[
            pl.BlockSpec((_TN, _H), lambda b: (b, 0)),
            pl.BlockSpec((_TN, _H), lambda b: (b, 0)),
            pl.BlockSpec((_TN, _H), lambda b: (b, 0)),
            pl.BlockSpec((_H, _H), lambda b: (0, 0)),
            pl.BlockSpec((1, _H), lambda b: (0, 0)),
        ],
        out_specs=pl.BlockSpec((_TN, _H), lambda b: (b, 0)),
        out_shape=jax.ShapeDtypeStruct((_N, _H), jnp.float32),
    )(h, a0, a1, Wl_eff, bl)


def _gate_body(h_ref, b2_ref, wg1_ref, bg1_ref, wg2_ref, bg2_ref, gate_ref, gmax_ref):
    t = jnp.dot(h_ref[...], wg1_ref[...], preferred_element_type=jnp.float32)
    t = _silu(t + bg1_ref[...])
    g = jnp.dot(t, wg2_ref[...], preferred_element_type=jnp.float32) + bg2_ref[...]
    gate_ref[...] = g
    mask = b2_ref[...] == lax.broadcasted_iota(jnp.int32, (_TN, _G), 1)
    tmax = jnp.max(jnp.where(mask, g, -1e30), axis=0, keepdims=True)

    @pl.when(pl.program_id(0) == 0)
    def _():
        gmax_ref[...] = jnp.full((1, _G), -1e30, jnp.float32)

    gmax_ref[...] = jnp.maximum(gmax_ref[...], tmax)


def _gate(h, batch2, Wg1, bg1, Wg2, bg2):
    return pl.pallas_call(
        _gate_body,
        grid=(_N // _TN,),
        in_specs=[
            pl.BlockSpec((_TN, _H), lambda b: (b, 0)),
            pl.BlockSpec((_TN, 1), lambda b: (b, 0)),
            pl.BlockSpec((_H, _H // 2), lambda b: (0, 0)),
            pl.BlockSpec((1, _H // 2), lambda b: (0, 0)),
            pl.BlockSpec((_H // 2, 1), lambda b: (0, 0)),
            pl.BlockSpec((1, 1), lambda b: (0, 0)),
        ],
        out_specs=[
            pl.BlockSpec((_TN, 1), lambda b: (b, 0)),
            pl.BlockSpec((1, _G), lambda b: (0, 0)),
        ],
        out_shape=[
            jax.ShapeDtypeStruct((_N, 1), jnp.float32),
            jax.ShapeDtypeStruct((1, _G), jnp.float32),
        ],
    )(h, batch2, Wg1, bg1, Wg2, bg2)


def _pool_body(h_ref, b2_ref, gate_ref, gmax_ref, num_ref, den_ref):
    mask = b2_ref[...] == lax.broadcasted_iota(jnp.int32, (_TN, _G), 1)
    gmax_n = jnp.max(jnp.where(mask, gmax_ref[...], -1e30), axis=1, keepdims=True)
    e = jnp.exp(gate_ref[...] - gmax_n)
    em = jnp.where(mask, e, 0.0)
    ntile = lax.dot_general(em, h_ref[...], (((0,), (0,)), ((), ())),
                            preferred_element_type=jnp.float32)
    ones = jnp.ones((_TN, 1), jnp.float32)
    dtile = lax.dot_general(em, ones, (((0,), (0,)), ((), ())),
                            preferred_element_type=jnp.float32)

    @pl.when(pl.program_id(0) == 0)
    def _():
        num_ref[...] = jnp.zeros((_G, _H), jnp.float32)
        den_ref[...] = jnp.zeros((_G, 1), jnp.float32)

    num_ref[...] += ntile
    den_ref[...] += dtile


def _pool(h, batch2, gate, gmax):
    return pl.pallas_call(
        _pool_body,
        grid=(_N // _TN,),
        in_specs=[
            pl.BlockSpec((_TN, _H), lambda b: (b, 0)),
            pl.BlockSpec((_TN, 1), lambda b: (b, 0)),
            pl.BlockSpec((_TN, 1), lambda b: (b, 0)),
            pl.BlockSpec((1, _G), lambda b: (0, 0)),
        ],
        out_specs=[
            pl.BlockSpec((_G, _H), lambda b: (0, 0)),
            pl.BlockSpec((_G, 1), lambda b: (0, 0)),
        ],
        out_shape=[
            jax.ShapeDtypeStruct((_G, _H), jnp.float32),
            jax.ShapeDtypeStruct((_G, 1), jnp.float32),
        ],
    )(h, batch2, gate, gmax)


def _final_body(num_ref, den_ref, wp_ref, bp_ref, g_ref, b_ref, o_ref):
    hg = num_ref[...] / (den_ref[...] + 1e-8)
    z = jnp.dot(hg, wp_ref[...], preferred_element_type=jnp.float32) + bp_ref[...]
    mu = jnp.mean(z, axis=-1, keepdims=True)
    var = jnp.mean((z - mu) ** 2, axis=-1, keepdims=True)
    zn = (z - mu) / jnp.sqrt(var + 1e-5) * g_ref[...] + b_ref[...]
    o_ref[...] = jax.nn.gelu(zn)


def _final(num, den, Wp, bp, ln_g, ln_b):
    return pl.pallas_call(
        _final_body,
        in_specs=[pl.BlockSpec(x.shape, lambda: tuple(0 for _ in x.shape))
                  for x in (num, den, Wp, bp, ln_g, ln_b)],
        out_specs=pl.BlockSpec((_G, _OUT), lambda: (0, 0)),
        out_shape=jax.ShapeDtypeStruct((_G, _OUT), jnp.float32),
    )(num, den, Wp, bp, ln_g, ln_b)


def kernel(node_features, edge_index, edge_dist, batch,
           W0, b0, Wf1, bf1, Wf2, bf2, Wl, bl,
           Wg1, bg1, Wg2, bg2, Wp, bp, ln_g, ln_b):
    src = jnp.pad(edge_index[0], (0, _EP - _E))
    dst = jnp.pad(edge_index[1], (0, _EP - _E), constant_values=_N)
    h = _embed(node_features, W0, b0.reshape(1, _H))
    Wf1d, bf1d, Wf2d, bf2d = _filter_weights(Wf1, bf1, Wf2, bf2)
    sc_msg = _make_sc_msg()
    P = jnp.zeros((_H, _H), jnp.float32).at[
        jnp.arange(_H), jnp.array(_PACK_PERM)].set(1.0)
    for i in range(_NI):
        fil = _filters_i(edge_dist, Wf1d[i], bf1d[i], Wf2d[i], bf2d[i])
        a0, a1 = sc_msg(h, fil, src, dst)
        h = _update(h, a0, a1, P @ Wl[i], bl[i].reshape(1, _H))
    batch2 = batch.reshape(_N, 1)
    gate, gmax = _gate(h, batch2, Wg1, bg1.reshape(1, _H // 2),
                       Wg2, bg2.reshape(1, 1))
    num, den = _pool(h, batch2, gate, gmax)
    return _final(num, den, Wp, bp.reshape(1, _OUT),
                  ln_g.reshape(1, _OUT), ln_b.reshape(1, _OUT))


# speedup vs baseline: 1.2519x; 1.2519x over previous
import functools

import jax
import jax.numpy as jnp
from jax import lax
from jax.experimental import pallas as pl
from jax.experimental.pallas import tpu as pltpu
from jax.experimental.pallas import tpu_sc as plsc

_N = 50000
_E = 800000
_H = 64
_RBF = 50
_G = 64
_OUT = 64
_NI = 3
_GAMMA = 10.0

_CH = 88
_NCH = 9216
_EP = _NCH * _CH
_TE = 2048
_TN = 2000
_NPAD = 50016
_NT = _NCH // 2 // 16
_ZROWS = _NPAD // 16
_OCH = 400
_NOCH = _N // _OCH

_PACK_PERM = [0] * _H
for _t in range(16):
    for _J in range(2):
        _PACK_PERM[32 * _J + 2 * _t] = 32 * _J + _t
        _PACK_PERM[32 * _J + 2 * _t + 1] = 32 * _J + 16 + _t
del _t, _J


def _silu(x):
    return x * jax.nn.sigmoid(x)


def _embed_body(x_ref, w_ref, b_ref, o_ref):
    t = jnp.dot(x_ref[...], w_ref[...], preferred_element_type=jnp.float32)
    o_ref[...] = _silu(t + b_ref[...])


def _embed(x, W0, b0):
    return pl.pallas_call(
        _embed_body,
        grid=(_N // _TN,),
        in_specs=[
            pl.BlockSpec((_TN, 3), lambda b: (b, 0)),
            pl.BlockSpec((3, _H), lambda b: (0, 0)),
            pl.BlockSpec((1, _H), lambda b: (0, 0)),
        ],
        out_specs=pl.BlockSpec((_TN, _H), lambda b: (b, 0)),
        out_shape=jax.ShapeDtypeStruct((_N, _H), jnp.float32),
    )(x, W0, b0)


def _filter_body(d_ref, wf1_ref, bf1_ref, wf2_ref, bf2_ref, o_ref):
    d2 = d_ref[...]
    col = lax.broadcasted_iota(jnp.int32, (1, 2 * _RBF), 1)
    centers = (col % _RBF).astype(jnp.float32) * (6.0 / (_RBF - 1))
    dsel = jnp.where(col >= _RBF, d2[:, 1:2], d2[:, 0:1])
    diff = dsel - centers
    rbf = jnp.exp(-_GAMMA * diff * diff)
    t = jnp.dot(rbf, wf1_ref[...], preferred_element_type=jnp.float32) + bf1_ref[...]
    t = _silu(t)
    o_ref[...] = (
        jnp.dot(t, wf2_ref[...], preferred_element_type=jnp.float32) + bf2_ref[...]
    )


def _filters_i(edge_dist, Wf1d, bf1d, Wf2d, bf2d):
    te2 = _TE // 2
    grid = (_E + _TE - 1) // _TE
    return pl.pallas_call(
        _filter_body,
        grid=(grid,),
        in_specs=[
            pl.BlockSpec((te2, 2), lambda b: (b, 0)),
            pl.BlockSpec((2 * _RBF, 2 * _H), lambda b: (0, 0)),
            pl.BlockSpec((1, 2 * _H), lambda b: (0, 0)),
            pl.BlockSpec((2 * _H, 2 * _H), lambda b: (0, 0)),
            pl.BlockSpec((1, 2 * _H), lambda b: (0, 0)),
        ],
        out_specs=pl.BlockSpec((te2, 2 * _H), lambda b: (b, 0)),
        out_shape=jax.ShapeDtypeStruct((_EP // 2, 2 * _H), jnp.float32),
    )(edge_dist.reshape(_E // 2, 2), Wf1d, bf1d, Wf2d, bf2d)


def _filter_weights(Wf1, bf1, Wf2, bf2):
    Wf1d = jnp.zeros((_NI, 2 * _RBF, 2 * _H), jnp.float32)
    Wf1d = Wf1d.at[:, :_RBF, :_H].set(Wf1).at[:, _RBF:, _H:].set(Wf1)
    Wf2d = jnp.zeros((_NI, 2 * _H, 2 * _H), jnp.float32)
    Wf2d = Wf2d.at[:, :_H, :_H].set(Wf2).at[:, _H:, _H:].set(Wf2)
    bf1d = jnp.tile(bf1, (1, 2)).reshape(_NI, 1, 2 * _H)
    bf2d = jnp.tile(bf2, (1, 2)).reshape(_NI, 1, 2 * _H)
    return Wf1d, bf1d, Wf2d, bf2d


@functools.lru_cache(maxsize=1)
def _make_sc_msg():
    mesh = plsc.VectorSubcoreMesh(core_axis_name="c", subcore_axis_name="s")

    @functools.partial(
        pl.kernel,
        out_type=[jax.ShapeDtypeStruct((_N, _H), jnp.bfloat16),
                  jax.ShapeDtypeStruct((_N, _H), jnp.bfloat16)],
        mesh=mesh,
        scratch_types=[
            pltpu.VMEM((4, _CH), jnp.int32),
            pltpu.VMEM((4, _CH), jnp.int32),
            pltpu.VMEM((2, _CH, _H), jnp.float32),
            pltpu.VMEM((2, _CH // 2, 2 * _H), jnp.float32),
            pltpu.VMEM((2, _CH, _H), jnp.bfloat16),
            pltpu.VMEM_SHARED((_NPAD, _H), jnp.bfloat16),
            pltpu.SemaphoreType.DMA, pltpu.SemaphoreType.DMA,
            pltpu.SemaphoreType.DMA, pltpu.SemaphoreType.DMA,
            pltpu.SemaphoreType.DMA, pltpu.SemaphoreType.DMA,
            pltpu.SemaphoreType.DMA, pltpu.SemaphoreType.DMA,
            pltpu.SemaphoreType.DMA, pltpu.SemaphoreType.DMA,
        ],
        compiler_params=pltpu.CompilerParams(use_tc_tiling_on_sc=False,
                                             needs_layout_passes=False),
    )
    def _sc_msg(h_hbm, fil_hbm, src_hbm, dst_hbm, agg0_hbm, agg1_hbm,
                src_v, dst_v, rows_v, fil_v, msg_v, acc_sh,
                is0, is1, is2, is3, fs0, fs1, gs0, gs1, ss0, ss1):
        c = lax.axis_index("c")
        s = lax.axis_index("s")
        isem = (is0, is1, is2, is3)
        fsem = (fs0, fs1)
        gsem = (gs0, gs1)
        ssem = (ss0, ss1)

        def chunk_of(i):
            return c * (_NCH // 2) + s + 16 * i

        def cbase_of(i):
            return jnp.minimum(chunk_of(i) * _CH, _E - _CH)

        def idx_load(b4, i):
            base = cbase_of(i)
            pltpu.async_copy(src_hbm.at[pl.ds(base, _CH)], src_v.at[b4], isem[b4])
            pltpu.async_copy(dst_hbm.at[pl.ds(base, _CH)], dst_v.at[b4], isem[b4])

        def idx_wait(b4):
            pltpu.make_async_copy(src_hbm.at[pl.ds(0, _CH)], src_v.at[b4], isem[b4]).wait()
            pltpu.make_async_copy(dst_hbm.at[pl.ds(0, _CH)], dst_v.at[b4], isem[b4]).wait()

        def fil_load(b2, i):
            fb = cbase_of(i) // 2
            pltpu.async_copy(fil_hbm.at[pl.ds(fb, _CH // 2)], fil_v.at[b2], fsem[b2])

        def fil_wait(b2):
            pltpu.make_async_copy(fil_hbm.at[pl.ds(0, _CH // 2)], fil_v.at[b2],
                                  fsem[b2]).wait()

        def scat_wait(b2, b4):
            pltpu.make_async_copy(msg_v.at[b2], acc_sh.at[dst_v.at[b4]],
                                  ssem[b2]).wait()

        def fire_gather(b2, b4):
            pltpu.async_copy(h_hbm.at[src_v.at[b4]], rows_v.at[b2], gsem[b2])

        def compute(b2, b4, i):
            delta = chunk_of(i) * _CH - cbase_of(i)

            @pl.when(delta > 0)
            def _():
                for g in range(_CH // 16):
                    sl = pl.ds(g * 16, 16)
                    m = (lax.iota(jnp.int32, 16) + (g * 16)) >= delta
                    dst_v[b4, sl] = jnp.where(m, dst_v[b4, sl], _N)

            fil_wait(b2)
            pltpu.make_async_copy(h_hbm.at[src_v.at[b4]], rows_v.at[b2],
                                  gsem[b2]).wait()

            @pl.loop(0, _CH // 2, unroll=2)
            def _mul(rr):
                for half in range(2):
                    r = 2 * rr + half
                    p = [rows_v[b2, r, pl.ds(j * 16, 16)]
                         * fil_v[b2, rr, pl.ds(half * _H + j * 16, 16)]
                         for j in range(4)]
                    msg_v[b2, r, pl.ds(0, 32)] = plsc.pack(
                        p[0], p[1], format=plsc.PackFormat.INTERLEAVED)
                    msg_v[b2, r, pl.ds(32, 32)] = plsc.pack(
                        p[2], p[3], format=plsc.PackFormat.INTERLEAVED)

            pltpu.async_copy(msg_v.at[b2], acc_sh.at[dst_v.at[b4]], ssem[b2],
                             add=True)

        idx_load(0, 0)
        idx_load(1, 1)
        fil_load(0, 0)

        @pl.loop(0, _CH)
        def _zero_rows(r):
            msg_v[1, r, pl.ds(0, 32)] = jnp.zeros((32,), jnp.bfloat16)
            msg_v[1, r, pl.ds(32, 32)] = jnp.zeros((32,), jnp.bfloat16)

        @pl.loop(0, _ZROWS // _CH)
        def _zero_acc(k):
            pltpu.sync_copy(msg_v.at[1],
                            acc_sh.at[pl.ds(s * _ZROWS + k * _CH, _CH)])
        pltpu.sync_copy(
            msg_v.at[1, pl.ds(0, _ZROWS % _CH)],
            acc_sh.at[pl.ds(s * _ZROWS + (_ZROWS // _CH) * _CH, _ZROWS % _CH)])

        plsc.subcore_barrier()

        idx_wait(0)
        fire_gather(0, 0)

        @pl.loop(0, _NT // 4)
        def _main(k):
            i0 = 4 * k
            for u in range(4):
                i = i0 + u
                b2, b4 = u % 2, u % 4
                nb2, nb4 = (u + 1) % 2, (u + 1) % 4

                @pl.when(i + 2 < _NT)
                def _():
                    idx_load((u + 2) % 4, i + 2)

                @pl.when(i + 1 < _NT)
                def _():
                    fil_load(nb2, i + 1)
                    idx_wait(nb4)

                    @pl.when(i >= 1)
                    def _():
                        scat_wait(nb2, nb4)

                    fire_gather(nb2, nb4)

                compute(b2, b4, i)

        scat_wait(0, 0)
        scat_wait(1, 1)
        plsc.subcore_barrier()

        @pl.when(c == 0)
        def _():
            @pl.loop(s, _NOCH, step=16)
            def _out(k):
                pltpu.sync_copy(acc_sh.at[pl.ds(k * _OCH, _OCH)],
                                agg0_hbm.at[pl.ds(k * _OCH, _OCH)])

        @pl.when(c == 1)
        def _():
            @pl.loop(s, _NOCH, step=16)
            def _out(k):
                pltpu.sync_copy(acc_sh.at[pl.ds(k * _OCH, _OCH)],
                                agg1_hbm.at[pl.ds(k * _OCH, _OCH)])

    return _sc_msg


def _update_body(h_ref, a0_ref, a1_ref, w_ref, b_ref, o_ref):
    a = a0_ref[...].astype(jnp.float32) + a1_ref[...].astype(jnp.float32)
    t = jnp.dot(a, w_ref[...], preferred_element_type=jnp.float32)
    o_ref[...] = h_ref[...] + _silu(t + b_ref[...])


def _update(h, a0, a1, Wl_eff, bl):
    return pl.pallas_call(
        _update_body,
        grid=(_N // _TN,),
        in_specs=[
            pl.BlockSpec((_TN, _H), lambda b: (b, 0)),
            pl.BlockSpec((_TN, _H), lambda b: (b, 0)),
            pl.BlockSpec((_TN, _H), lambda b: (b, 0)),
            pl.BlockSpec((_H, _H), lambda b: (0, 0)),
            pl.BlockSpec((1, _H), lambda b: (0, 0)),
        ],
        out_specs=pl.BlockSpec((_TN, _H), lambda b: (b, 0)),
        out_shape=jax.ShapeDtypeStruct((_N, _H), jnp.float32),
    )(h, a0, a1, Wl_eff, bl)


def _gate_body(h_ref, b2_ref, wg1_ref, bg1_ref, wg2_ref, bg2_ref, gate_ref, gmax_ref):
    t = jnp.dot(h_ref[...], wg1_ref[...], preferred_element_type=jnp.float32)
    t = _silu(t + bg1_ref[...])
    g = jnp.dot(t, wg2_ref[...], preferred_element_type=jnp.float32) + bg2_ref[...]
    gate_ref[...] = g
    mask = b2_ref[...] == lax.broadcasted_iota(jnp.int32, (_TN, _G), 1)
    tmax = jnp.max(jnp.where(mask, g, -1e30), axis=0, keepdims=True)

    @pl.when(pl.program_id(0) == 0)
    def _():
        gmax_ref[...] = jnp.full((1, _G), -1e30, jnp.float32)

    gmax_ref[...] = jnp.maximum(gmax_ref[...], tmax)


def _gate(h, batch2, Wg1, bg1, Wg2, bg2):
    return pl.pallas_call(
        _gate_body,
        grid=(_N // _TN,),
        in_specs=[
            pl.BlockSpec((_TN, _H), lambda b: (b, 0)),
            pl.BlockSpec((_TN, 1), lambda b: (b, 0)),
            pl.BlockSpec((_H, _H // 2), lambda b: (0, 0)),
            pl.BlockSpec((1, _H // 2), lambda b: (0, 0)),
            pl.BlockSpec((_H // 2, 1), lambda b: (0, 0)),
            pl.BlockSpec((1, 1), lambda b: (0, 0)),
        ],
        out_specs=[
            pl.BlockSpec((_TN, 1), lambda b: (b, 0)),
            pl.BlockSpec((1, _G), lambda b: (0, 0)),
        ],
        out_shape=[
            jax.ShapeDtypeStruct((_N, 1), jnp.float32),
            jax.ShapeDtypeStruct((1, _G), jnp.float32),
        ],
    )(h, batch2, Wg1, bg1, Wg2, bg2)


def _pool_body(h_ref, b2_ref, gate_ref, gmax_ref, num_ref, den_ref):
    mask = b2_ref[...] == lax.broadcasted_iota(jnp.int32, (_TN, _G), 1)
    gmax_n = jnp.max(jnp.where(mask, gmax_ref[...], -1e30), axis=1, keepdims=True)
    e = jnp.exp(gate_ref[...] - gmax_n)
    em = jnp.where(mask, e, 0.0)
    ntile = lax.dot_general(em, h_ref[...], (((0,), (0,)), ((), ())),
                            preferred_element_type=jnp.float32)
    ones = jnp.ones((_TN, 1), jnp.float32)
    dtile = lax.dot_general(em, ones, (((0,), (0,)), ((), ())),
                            preferred_element_type=jnp.float32)

    @pl.when(pl.program_id(0) == 0)
    def _():
        num_ref[...] = jnp.zeros((_G, _H), jnp.float32)
        den_ref[...] = jnp.zeros((_G, 1), jnp.float32)

    num_ref[...] += ntile
    den_ref[...] += dtile


def _pool(h, batch2, gate, gmax):
    return pl.pallas_call(
        _pool_body,
        grid=(_N // _TN,),
        in_specs=[
            pl.BlockSpec((_TN, _H), lambda b: (b, 0)),
            pl.BlockSpec((_TN, 1), lambda b: (b, 0)),
            pl.BlockSpec((_TN, 1), lambda b: (b, 0)),
            pl.BlockSpec((1, _G), lambda b: (0, 0)),
        ],
        out_specs=[
            pl.BlockSpec((_G, _H), lambda b: (0, 0)),
            pl.BlockSpec((_G, 1), lambda b: (0, 0)),
        ],
        out_shape=[
            jax.ShapeDtypeStruct((_G, _H), jnp.float32),
            jax.ShapeDtypeStruct((_G, 1), jnp.float32),
        ],
    )(h, batch2, gate, gmax)


def _final_body(num_ref, den_ref, wp_ref, bp_ref, g_ref, b_ref, o_ref):
    hg = num_ref[...] / (den_ref[...] + 1e-8)
    z = jnp.dot(hg, wp_ref[...], preferred_element_type=jnp.float32) + bp_ref[...]
    mu = jnp.mean(z, axis=-1, keepdims=True)
    var = jnp.mean((z - mu) ** 2, axis=-1, keepdims=True)
    zn = (z - mu) / jnp.sqrt(var + 1e-5) * g_ref[...] + b_ref[...]
    o_ref[...] = jax.nn.gelu(zn)


def _final(num, den, Wp, bp, ln_g, ln_b):
    return pl.pallas_call(
        _final_body,
        in_specs=[pl.BlockSpec(x.shape, lambda: tuple(0 for _ in x.shape))
                  for x in (num, den, Wp, bp, ln_g, ln_b)],
        out_specs=pl.BlockSpec((_G, _OUT), lambda: (0, 0)),
        out_shape=jax.ShapeDtypeStruct((_G, _OUT), jnp.float32),
    )(num, den, Wp, bp, ln_g, ln_b)


def kernel(node_features, edge_index, edge_dist, batch,
           W0, b0, Wf1, bf1, Wf2, bf2, Wl, bl,
           Wg1, bg1, Wg2, bg2, Wp, bp, ln_g, ln_b):
    src = edge_index[0]
    dst = edge_index[1]
    h = _embed(node_features, W0, b0.reshape(1, _H))
    Wf1d, bf1d, Wf2d, bf2d = _filter_weights(Wf1, bf1, Wf2, bf2)
    sc_msg = _make_sc_msg()
    P = jnp.zeros((_H, _H), jnp.float32).at[
        jnp.arange(_H), jnp.array(_PACK_PERM)].set(1.0)
    for i in range(_NI):
        fil = _filters_i(edge_dist, Wf1d[i], bf1d[i], Wf2d[i], bf2d[i])
        a0, a1 = sc_msg(h, fil, src, dst)
        h = _update(h, a0, a1, P @ Wl[i], bl[i].reshape(1, _H))
    batch2 = batch.reshape(_N, 1)
    gate, gmax = _gate(h, batch2, Wg1, bg1.reshape(1, _H // 2),
                       Wg2, bg2.reshape(1, 1))
    num, den = _pool(h, batch2, gate, gmax)
    return _final(num, den, Wp, bp.reshape(1, _OUT),
                  ln_g.reshape(1, _OUT), ln_b.reshape(1, _OUT))
